# baseline (speedup 1.0000x reference)
import functools

import jax
import jax.numpy as jnp
from jax import lax
from jax.experimental import pallas as pl
from jax.experimental.pallas import tpu as pltpu
from jax.experimental.pallas import tpu_sc as plsc

NUM_CLASSES = 100000
FEAT = 64
BATCH = 16384
NUM_CORES = 2
NUM_SUBCORES = 16
LANES = 16
NW = NUM_CORES * NUM_SUBCORES
BPW = BATCH // NW
GCHUNK = 128
NCHUNK = BPW // GCHUNK
GROUPS = BPW // LANES


def _sc_partials(x, labels3, centers):
    mesh = plsc.VectorSubcoreMesh(core_axis_name="c", subcore_axis_name="s")

    @functools.partial(
        pl.kernel,
        mesh=mesh,
        out_type=jax.ShapeDtypeStruct((NW, LANES), jnp.float32),
        compiler_params=pltpu.CompilerParams(
            needs_layout_passes=False, use_tc_tiling_on_sc=False
        ),
        scratch_types=[
            pltpu.VMEM((NCHUNK, GCHUNK), jnp.int32),
            pltpu.VMEM((BPW * FEAT,), jnp.float32),
            pltpu.VMEM((BPW, FEAT), jnp.float32),
            pltpu.VMEM((LANES * LANES,), jnp.float32),
            pltpu.VMEM((LANES,), jnp.float32),
            pltpu.SemaphoreType.DMA,
        ],
    )
    def k(x_hbm, lab_hbm, cen_hbm, out_hbm, idx_v, x_v, c_v, tmp_v, acc_v, sem):
        wid = lax.axis_index("s") * NUM_CORES + lax.axis_index("c")
        base = wid * BPW * FEAT

        pltpu.sync_copy(lab_hbm.at[wid], idx_v)
        gathers = [
            pltpu.async_copy(
                cen_hbm.at[idx_v.at[j]],
                c_v.at[pl.ds(j * GCHUNK, GCHUNK)],
                sem,
            )
            for j in range(NCHUNK)
        ]
        pltpu.sync_copy(x_hbm.at[pl.ds(base, BPW * FEAT)], x_v)
        for g in gathers:
            g.wait()

        lane_iota = lax.iota(jnp.int32, LANES)

        def group_body(g, acc):
            for r in range(LANES):
                row = g * LANES + r
                s4 = jnp.zeros((LANES,), jnp.float32)
                for kk in range(FEAT // LANES):
                    xv = x_v[pl.ds(row * FEAT + kk * LANES, LANES)]
                    cv = c_v[row, pl.ds(kk * LANES, LANES)]
                    d = xv - cv
                    s4 = s4 + d * d
                tmp_v[pl.ds(r * LANES, LANES)] = s4
            s = jnp.zeros((LANES,), jnp.float32)
            for j in range(LANES):
                s = s + plsc.load_gather(tmp_v, [lane_iota * LANES + j])
            dist = jnp.minimum(jnp.maximum(s, 1e-12), 1e12)
            return acc + dist

        acc = lax.fori_loop(0, GROUPS, group_body, jnp.zeros((LANES,), jnp.float32))
        acc_v[...] = acc
        pltpu.sync_copy(acc_v, out_hbm.at[wid])

    return k(x, labels3, centers)


def _tc_reduce(partials):
    def body(p_ref, o_ref):
        o_ref[0, 0] = jnp.sum(p_ref[...]) * (1.0 / BATCH)

    return pl.pallas_call(
        body,
        out_shape=jax.ShapeDtypeStruct((1, 1), jnp.float32),
        out_specs=pl.BlockSpec(memory_space=pltpu.SMEM),
    )(partials)


def kernel(x, labels, centers):
    labels3 = labels.astype(jnp.int32).reshape(NW, NCHUNK, GCHUNK)
    partials = _sc_partials(x.reshape(-1), labels3, centers)
    return _tc_reduce(partials)[0, 0]


# speedup vs baseline: 1.0022x; 1.0022x over previous
import functools

import jax
import jax.numpy as jnp
from jax import lax
from jax.experimental import pallas as pl
from jax.experimental.pallas import tpu as pltpu
from jax.experimental.pallas import tpu_sc as plsc

NUM_CLASSES = 100000
FEAT = 64
BATCH = 16384
NUM_CORES = 2
NUM_SUBCORES = 16
LANES = 16
NW = NUM_CORES * NUM_SUBCORES
BPW = BATCH // NW
GCHUNK = 128
NCHUNK = BPW // GCHUNK
GROUPS = BPW // LANES


def _sc_partials(x, labels3, centers):
    mesh = plsc.VectorSubcoreMesh(core_axis_name="c", subcore_axis_name="s")

    @functools.partial(
        pl.kernel,
        mesh=mesh,
        out_type=jax.ShapeDtypeStruct((NW, LANES), jnp.float32),
        compiler_params=pltpu.CompilerParams(
            needs_layout_passes=False, use_tc_tiling_on_sc=False
        ),
        scratch_types=[
            pltpu.VMEM((NCHUNK, GCHUNK), jnp.int32),
            pltpu.VMEM((BPW, FEAT), jnp.float32),
            pltpu.VMEM((BPW, FEAT), jnp.float32),
            pltpu.VMEM((LANES * LANES,), jnp.float32),
            pltpu.VMEM((LANES,), jnp.float32),
            pltpu.SemaphoreType.DMA,
        ],
    )
    def k(x_hbm, lab_hbm, cen_hbm, out_hbm, idx_v, x_v, c_v, tmp_v, acc_v, sem):
        wid = lax.axis_index("s") * NUM_CORES + lax.axis_index("c")
        base = wid * BPW

        pltpu.sync_copy(lab_hbm.at[wid], idx_v)
        gathers = [
            pltpu.async_copy(
                cen_hbm.at[idx_v.at[j]],
                c_v.at[pl.ds(j * GCHUNK, GCHUNK)],
                sem,
            )
            for j in range(NCHUNK)
        ]
        pltpu.sync_copy(x_hbm.at[pl.ds(base, BPW)], x_v)
        for g in gathers:
            g.wait()

        lane_iota = lax.iota(jnp.int32, LANES)

        def group_body(g, acc):
            for r in range(LANES):
                row = g * LANES + r
                s4 = jnp.zeros((LANES,), jnp.float32)
                for kk in range(FEAT // LANES):
                    xv = x_v[row, pl.ds(kk * LANES, LANES)]
                    cv = c_v[row, pl.ds(kk * LANES, LANES)]
                    d = xv - cv
                    s4 = s4 + d * d
                tmp_v[pl.ds(r * LANES, LANES)] = s4
            s = jnp.zeros((LANES,), jnp.float32)
            for j in range(LANES):
                s = s + plsc.load_gather(tmp_v, [lane_iota * LANES + j])
            dist = jnp.minimum(jnp.maximum(s, 1e-12), 1e12)
            return acc + dist

        acc = lax.fori_loop(0, GROUPS, group_body, jnp.zeros((LANES,), jnp.float32))
        acc_v[...] = acc
        pltpu.sync_copy(acc_v, out_hbm.at[wid])

    return k(x, labels3, centers)


def _tc_reduce(partials):
    def body(p_ref, o_ref):
        o_ref[0, 0] = jnp.sum(p_ref[...]) * (1.0 / BATCH)

    return pl.pallas_call(
        body,
        out_shape=jax.ShapeDtypeStruct((1, 1), jnp.float32),
        out_specs=pl.BlockSpec(memory_space=pltpu.SMEM),
    )(partials)


def kernel(x, labels, centers):
    labels3 = labels.astype(jnp.int32).reshape(NW, NCHUNK, GCHUNK)
    partials = _sc_partials(x, labels3, centers)
    return _tc_reduce(partials)[0, 0]


# speedup vs baseline: 1.4257x; 1.4226x over previous
import functools

import jax
import jax.numpy as jnp
from jax import lax
from jax.experimental import pallas as pl
from jax.experimental.pallas import tpu as pltpu
from jax.experimental.pallas import tpu_sc as plsc

NUM_CLASSES = 100000
FEAT = 64
BATCH = 16384
NUM_CORES = 2
NUM_SUBCORES = 16
LANES = 16
NW = NUM_CORES * NUM_SUBCORES
FPW = FEAT // NW
CHUNK = 4096
NCHUNKS = BATCH // CHUNK
GROUPS = CHUNK // LANES


def _sc_partials(xt, labels, cent):
    mesh = plsc.VectorSubcoreMesh(core_axis_name="c", subcore_axis_name="s")

    @functools.partial(
        pl.kernel,
        mesh=mesh,
        out_type=jax.ShapeDtypeStruct((NW, BATCH), jnp.float32),
        compiler_params=pltpu.CompilerParams(
            needs_layout_passes=False, use_tc_tiling_on_sc=True
        ),
        scratch_types=[
            pltpu.VMEM((NUM_CLASSES,), jnp.float32),
            pltpu.VMEM((CHUNK,), jnp.int32),
            pltpu.VMEM((CHUNK,), jnp.float32),
            pltpu.VMEM((BATCH,), jnp.float32),
        ],
    )
    def k(xt_hbm, lab_hbm, cen_hbm, out_hbm, tab_v, lab_v, x_v, acc_v):
        wid = lax.axis_index("s") * NUM_CORES + lax.axis_index("c")

        for p in range(FPW):
            f = wid + p * NW
            pltpu.sync_copy(cen_hbm.at[f], tab_v)
            for ch in range(NCHUNKS):
                pltpu.sync_copy(lab_hbm.at[pl.ds(ch * CHUNK, CHUNK)], lab_v)
                pltpu.sync_copy(xt_hbm.at[f, pl.ds(ch * CHUNK, CHUNK)], x_v)

                def group_body(g, _, ch=ch, p=p):
                    off = g * LANES
                    idx = lab_v[pl.ds(off, LANES)]
                    cg = plsc.load_gather(tab_v, [idx])
                    xv = x_v[pl.ds(off, LANES)]
                    d = xv - cg
                    d2 = d * d
                    aoff = ch * CHUNK + off
                    if p == 0:
                        acc_v[pl.ds(aoff, LANES)] = d2
                    else:
                        acc_v[pl.ds(aoff, LANES)] = acc_v[pl.ds(aoff, LANES)] + d2
                    return 0

                lax.fori_loop(0, GROUPS, group_body, 0)
        pltpu.sync_copy(acc_v, out_hbm.at[wid])

    return k(xt, labels, cent)


def _tc_reduce(partials):
    def body(p_ref, o_ref):
        dist = jnp.sum(p_ref[...], axis=0)
        dist = jnp.minimum(jnp.maximum(dist, 1e-12), 1e12)
        o_ref[0, 0] = jnp.sum(dist) * (1.0 / BATCH)

    return pl.pallas_call(
        body,
        out_shape=jax.ShapeDtypeStruct((1, 1), jnp.float32),
        out_specs=pl.BlockSpec(memory_space=pltpu.SMEM),
    )(partials)


def kernel(x, labels, centers):
    partials = _sc_partials(x.T, labels.astype(jnp.int32), centers.T)
    return _tc_reduce(partials)[0, 0]


# speedup vs baseline: 1.9087x; 1.3387x over previous
import functools

import jax
import jax.numpy as jnp
from jax import lax
from jax.experimental import pallas as pl
from jax.experimental.pallas import tpu as pltpu
from jax.experimental.pallas import tpu_sc as plsc

NUM_CLASSES = 100000
FEAT = 64
BATCH = 16384
NUM_CORES = 2
NUM_SUBCORES = 16
LANES = 16
NW = NUM_CORES * NUM_SUBCORES
FPW = FEAT // NW
CHUNK = 4096
NCHUNKS = BATCH // CHUNK
GROUPS = CHUNK // LANES


def _sc_partials(xt, labels, cent):
    mesh = plsc.VectorSubcoreMesh(core_axis_name="c", subcore_axis_name="s")

    @functools.partial(
        pl.kernel,
        mesh=mesh,
        out_type=jax.ShapeDtypeStruct((NW, BATCH), jnp.float32),
        compiler_params=pltpu.CompilerParams(
            needs_layout_passes=False, use_tc_tiling_on_sc=True
        ),
        scratch_types=[
            pltpu.VMEM((NUM_CLASSES,), jnp.float32),
            pltpu.VMEM((CHUNK,), jnp.int32),
            pltpu.VMEM((CHUNK,), jnp.float32),
            pltpu.VMEM((BATCH,), jnp.float32),
        ],
    )
    def k(xt_hbm, lab_hbm, cen_hbm, out_hbm, tab_v, lab_v, x_v, acc_v):
        wid = lax.axis_index("s") * NUM_CORES + lax.axis_index("c")

        for p in range(FPW):
            f = wid + p * NW
            pltpu.sync_copy(cen_hbm.at[f], tab_v)
            for ch in range(NCHUNKS):
                pltpu.sync_copy(lab_hbm.at[pl.ds(ch * CHUNK, CHUNK)], lab_v)
                pltpu.sync_copy(xt_hbm.at[f, pl.ds(ch * CHUNK, CHUNK)], x_v)

                @plsc.parallel_loop(0, GROUPS, unroll=8)
                def group_body(g, ch=ch, p=p):
                    off = g * LANES
                    idx = lab_v[pl.ds(off, LANES)]
                    cg = plsc.load_gather(tab_v, [idx])
                    xv = x_v[pl.ds(off, LANES)]
                    d = xv - cg
                    d2 = d * d
                    aoff = ch * CHUNK + off
                    if p == 0:
                        acc_v[pl.ds(aoff, LANES)] = d2
                    else:
                        acc_v[pl.ds(aoff, LANES)] = acc_v[pl.ds(aoff, LANES)] + d2
        pltpu.sync_copy(acc_v, out_hbm.at[wid])

    return k(xt, labels, cent)


def _tc_reduce(partials):
    def body(p_ref, o_ref):
        dist = jnp.sum(p_ref[...], axis=0)
        dist = jnp.minimum(jnp.maximum(dist, 1e-12), 1e12)
        o_ref[0, 0] = jnp.sum(dist) * (1.0 / BATCH)

    return pl.pallas_call(
        body,
        out_shape=jax.ShapeDtypeStruct((1, 1), jnp.float32),
        out_specs=pl.BlockSpec(memory_space=pltpu.SMEM),
    )(partials)


def kernel(x, labels, centers):
    partials = _sc_partials(x.T, labels.astype(jnp.int32), centers.T)
    return _tc_reduce(partials)[0, 0]


# speedup vs baseline: 2.0032x; 1.0496x over previous
import functools

import jax
import jax.numpy as jnp
from jax import lax
from jax.experimental import pallas as pl
from jax.experimental.pallas import tpu as pltpu
from jax.experimental.pallas import tpu_sc as plsc

NUM_CLASSES = 100000
FEAT = 64
BATCH = 16384
NUM_CORES = 2
NUM_SUBCORES = 16
LANES = 16
NW = NUM_CORES * NUM_SUBCORES
FPW = FEAT // NW
CHUNK = 2048
NCHUNKS = BATCH // CHUNK
GROUPS = CHUNK // LANES


def _sc_partials(xt, labels, cent):
    mesh = plsc.VectorSubcoreMesh(core_axis_name="c", subcore_axis_name="s")

    @functools.partial(
        pl.kernel,
        mesh=mesh,
        out_type=jax.ShapeDtypeStruct((NW, BATCH), jnp.float32),
        compiler_params=pltpu.CompilerParams(
            needs_layout_passes=False, use_tc_tiling_on_sc=True
        ),
        scratch_types=[
            pltpu.VMEM((NUM_CLASSES,), jnp.float32),
            pltpu.VMEM((2, CHUNK), jnp.int32),
            pltpu.VMEM((2, CHUNK), jnp.float32),
            pltpu.VMEM((BATCH,), jnp.float32),
            pltpu.SemaphoreType.DMA,
            pltpu.SemaphoreType.DMA,
            pltpu.SemaphoreType.DMA,
        ],
    )
    def k(xt_hbm, lab_hbm, cen_hbm, out_hbm, tab_v, lab_v, x_v, acc_v,
          sem_t, sem0, sem1):
        wid = lax.axis_index("s") * NUM_CORES + lax.axis_index("c")
        sems = (sem0, sem1)

        for p in range(FPW):
            f = wid + p * NW
            tab_cp = pltpu.async_copy(cen_hbm.at[f], tab_v, sem_t)
            pending = [
                pltpu.async_copy(
                    lab_hbm.at[pl.ds(0, CHUNK)], lab_v.at[0], sems[0]
                ),
                pltpu.async_copy(
                    xt_hbm.at[f, pl.ds(0, CHUNK)], x_v.at[0], sems[0]
                ),
            ]
            tab_cp.wait()
            for ch in range(NCHUNKS):
                buf = ch % 2
                nbuf = (ch + 1) % 2
                if ch + 1 < NCHUNKS:
                    nxt = [
                        pltpu.async_copy(
                            lab_hbm.at[pl.ds((ch + 1) * CHUNK, CHUNK)],
                            lab_v.at[nbuf], sems[nbuf],
                        ),
                        pltpu.async_copy(
                            xt_hbm.at[f, pl.ds((ch + 1) * CHUNK, CHUNK)],
                            x_v.at[nbuf], sems[nbuf],
                        ),
                    ]
                else:
                    nxt = []
                for cp in pending:
                    cp.wait()
                pending = nxt

                @plsc.parallel_loop(0, GROUPS, unroll=8)
                def group_body(g, ch=ch, p=p, buf=buf):
                    off = g * LANES
                    idx = lab_v[buf, pl.ds(off, LANES)]
                    cg = plsc.load_gather(tab_v, [idx])
                    xv = x_v[buf, pl.ds(off, LANES)]
                    d = xv - cg
                    d2 = d * d
                    aoff = ch * CHUNK + off
                    if p == 0:
                        acc_v[pl.ds(aoff, LANES)] = d2
                    else:
                        acc_v[pl.ds(aoff, LANES)] = acc_v[pl.ds(aoff, LANES)] + d2
        pltpu.sync_copy(acc_v, out_hbm.at[wid])

    return k(xt, labels, cent)


def _tc_reduce(partials):
    def body(p_ref, o_ref):
        dist = jnp.sum(p_ref[...], axis=0)
        dist = jnp.minimum(jnp.maximum(dist, 1e-12), 1e12)
        o_ref[0, 0] = jnp.sum(dist) * (1.0 / BATCH)

    return pl.pallas_call(
        body,
        out_shape=jax.ShapeDtypeStruct((1, 1), jnp.float32),
        out_specs=pl.BlockSpec(memory_space=pltpu.SMEM),
    )(partials)


def kernel(x, labels, centers):
    partials = _sc_partials(x.T, labels.astype(jnp.int32), centers.T)
    return _tc_reduce(partials)[0, 0]


# speedup vs baseline: 2.0086x; 1.0027x over previous
import functools

import jax
import jax.numpy as jnp
from jax import lax
from jax.experimental import pallas as pl
from jax.experimental.pallas import tpu as pltpu
from jax.experimental.pallas import tpu_sc as plsc

NUM_CLASSES = 100000
FEAT = 64
BATCH = 16384
NUM_CORES = 2
NUM_SUBCORES = 16
LANES = 16
NW = NUM_CORES * NUM_SUBCORES
FPW = FEAT // NW
CHUNK = 2048
NCHUNKS = BATCH // CHUNK
GROUPS = CHUNK // LANES


def _sc_partials(xt, labels, cent):
    mesh = plsc.VectorSubcoreMesh(core_axis_name="c", subcore_axis_name="s")

    @functools.partial(
        pl.kernel,
        mesh=mesh,
        out_type=jax.ShapeDtypeStruct((NW, BATCH), jnp.float32),
        compiler_params=pltpu.CompilerParams(
            needs_layout_passes=False, use_tc_tiling_on_sc=True
        ),
        scratch_types=[
            pltpu.VMEM((NUM_CLASSES,), jnp.float32),
            pltpu.VMEM((2, CHUNK), jnp.int32),
            pltpu.VMEM((2, CHUNK), jnp.float32),
            pltpu.VMEM((BATCH,), jnp.float32),
            pltpu.SemaphoreType.DMA,
            pltpu.SemaphoreType.DMA,
            pltpu.SemaphoreType.DMA,
        ],
    )
    def k(xt_hbm, lab_hbm, cen_hbm, out_hbm, tab_v, lab_v, x_v, acc_v,
          sem_t, sem0, sem1):
        wid = lax.axis_index("s") * NUM_CORES + lax.axis_index("c")
        sems = (sem0, sem1)

        for p in range(FPW):
            f = wid + p * NW
            with jax.named_scope(f"tabload{p}"):
                tab_cp = pltpu.async_copy(cen_hbm.at[f], tab_v, sem_t)
                pending = [
                    pltpu.async_copy(
                        lab_hbm.at[pl.ds(0, CHUNK)], lab_v.at[0], sems[0]
                    ),
                    pltpu.async_copy(
                        xt_hbm.at[f, pl.ds(0, CHUNK)], x_v.at[0], sems[0]
                    ),
                ]
                tab_cp.wait()
            compute_scope = jax.named_scope(f"chunks{p}")
            compute_scope.__enter__()
            for ch in range(NCHUNKS):
                buf = ch % 2
                nbuf = (ch + 1) % 2
                if ch + 1 < NCHUNKS:
                    nxt = [
                        pltpu.async_copy(
                            lab_hbm.at[pl.ds((ch + 1) * CHUNK, CHUNK)],
                            lab_v.at[nbuf], sems[nbuf],
                        ),
                        pltpu.async_copy(
                            xt_hbm.at[f, pl.ds((ch + 1) * CHUNK, CHUNK)],
                            x_v.at[nbuf], sems[nbuf],
                        ),
                    ]
                else:
                    nxt = []
                for cp in pending:
                    cp.wait()
                pending = nxt

                @plsc.parallel_loop(0, GROUPS, unroll=8)
                def group_body(g, ch=ch, p=p, buf=buf):
                    off = g * LANES
                    idx = lab_v[buf, pl.ds(off, LANES)]
                    cg = plsc.load_gather(tab_v, [idx])
                    xv = x_v[buf, pl.ds(off, LANES)]
                    d = xv - cg
                    d2 = d * d
                    aoff = ch * CHUNK + off
                    if p == 0:
                        acc_v[pl.ds(aoff, LANES)] = d2
                    else:
                        acc_v[pl.ds(aoff, LANES)] = acc_v[pl.ds(aoff, LANES)] + d2
            compute_scope.__exit__(None, None, None)
        pltpu.sync_copy(acc_v, out_hbm.at[wid])

    return k(xt, labels, cent)


def _tc_reduce(partials):
    def body(p_ref, o_ref):
        dist = jnp.sum(p_ref[...], axis=0)
        dist = jnp.minimum(jnp.maximum(dist, 1e-12), 1e12)
        o_ref[0, 0] = jnp.sum(dist) * (1.0 / BATCH)

    return pl.pallas_call(
        body,
        out_shape=jax.ShapeDtypeStruct((1, 1), jnp.float32),
        out_specs=pl.BlockSpec(memory_space=pltpu.SMEM),
    )(partials)


def kernel(x, labels, centers):
    partials = _sc_partials(x.T, labels.astype(jnp.int32), centers.T)
    return _tc_reduce(partials)[0, 0]
